# baseline (speedup 1.0000x reference)
import functools

import jax
import jax.numpy as jnp
from jax import lax
from jax.experimental import pallas as pl
from jax.experimental.pallas import tpu as pltpu
from jax.experimental.pallas import tpu_sc as plsc

_NU = 25000
_NI = 20000
_NA = 5000
_NN = 50000
_E = 800000
_D = 64
_BATCH = 4096
_DECAY = 1e-4

_NC = 2
_NS = 16
_HALF = 25088
_NPAD = _NC * _HALF
_TRASH = _HALF
_ACC_ROWS = _HALF + 128
_ZROWS = _ACC_ROWS // _NS
_WROWS = _HALF // _NS
_EBLK = 256
_NBLK = 196
_ET = _EBLK * _NBLK
_EPAD = _ET * _NS
_EROWS = _EPAD // 128


def _prop_body(x, srcr, dstr, wr, y, src_v, dst_v, dloc_v, w_v, rows,
               acc, sem):
    c = lax.axis_index("c")
    s = lax.axis_index("s")
    base = c * _HALF
    zero16 = jnp.zeros((16,), jnp.float32)
    iota = lax.iota(jnp.int32, 16)

    def _zb(r, carry):
        for q in range(4):
            rows[r, pl.ds(q * 16, 16)] = zero16
        return carry

    lax.fori_loop(0, _EBLK, _zb, 0)
    for k in range(6):
        pltpu.sync_copy(rows, acc.at[pl.ds(s * _ZROWS + k * _EBLK, _EBLK)])
    pltpu.sync_copy(rows.at[pl.ds(0, _ZROWS - 6 * _EBLK)],
                    acc.at[pl.ds(s * _ZROWS + 6 * _EBLK, _ZROWS - 6 * _EBLK)])
    plsc.subcore_barrier()

    def _blk(b, carry):
        row0 = s * (_NBLK * 2) + b * 2
        pltpu.sync_copy(srcr.at[pl.ds(row0, 2)], src_v)
        pltpu.sync_copy(dstr.at[pl.ds(row0, 2)], dst_v)
        pltpu.sync_copy(wr.at[pl.ds(row0, 2)], w_v)
        cps = [
            pltpu.async_copy(x.at[src_v.at[j]],
                             rows.at[pl.ds(j * 128, 128)], sem)
            for j in range(2)
        ]
        for j in range(2):
            for g8 in range(8):
                off = g8 * 16
                d16 = dst_v[j, pl.ds(off, 16)]
                lidx = d16 - base
                m = (lidx >= 0) & (lidx < _HALF)
                spread = _TRASH + (d16 & 127)
                dloc_v[j, pl.ds(off, 16)] = jnp.where(m, lidx, spread)
        for cp in cps:
            cp.wait()
        for j in range(2):
            def _sc(g8, carry, j=j):
                w16 = w_v[j, pl.ds(g8 * 16, 16)]
                rowi = iota + (j * 128 + g8 * 16)
                dimv = jnp.zeros((16,), jnp.int32)
                for d in range(64):
                    v = plsc.load_gather(rows, [rowi, dimv])
                    plsc.store_scatter(rows, [rowi, dimv], v * w16)
                    dimv = dimv + 1
                return carry

            lax.fori_loop(0, 8, _sc, 0)
        for j in range(2):
            pltpu.sync_copy(rows.at[pl.ds(j * 128, 128)],
                            acc.at[dloc_v.at[j]], add=True)
        return carry

    lax.fori_loop(0, _NBLK, _blk, 0)
    plsc.subcore_barrier()
    pltpu.sync_copy(acc.at[pl.ds(s * _WROWS, _WROWS)],
                    y.at[pl.ds(c * _HALF + s * _WROWS, _WROWS)])


_prop = pl.kernel(
    _prop_body,
    out_type=jax.ShapeDtypeStruct((_NPAD, _D), jnp.float32),
    mesh=plsc.VectorSubcoreMesh(core_axis_name="c", subcore_axis_name="s"),
    compiler_params=pltpu.CompilerParams(
        needs_layout_passes=False, use_tc_tiling_on_sc=False),
    scratch_types=[
        pltpu.VMEM((2, 128), jnp.int32),
        pltpu.VMEM((2, 128), jnp.int32),
        pltpu.VMEM((2, 128), jnp.int32),
        pltpu.VMEM((2, 128), jnp.float32),
        pltpu.VMEM((_EBLK, _D), jnp.float32),
        pltpu.VMEM_SHARED((_ACC_ROWS, _D), jnp.float32),
        pltpu.SemaphoreType.DMA,
    ],
)


def _score_body(x0, x1, x2, x3, u2d, p2d, n2d, diff_out, reg_out,
                u_all, p_all, n_all, ub, pb, nb, diff_v, regv, sem):
    c = lax.axis_index("c")
    s = lax.axis_index("s")
    wid = s * _NC + c
    iota = lax.iota(jnp.int32, 16)
    zero16 = jnp.zeros((16,), jnp.float32)

    pltpu.sync_copy(u2d, u_all)
    pltpu.sync_copy(p2d, p_all)
    pltpu.sync_copy(n2d, n_all)
    cps = [pltpu.async_copy(x0.at[u_all.at[wid]], ub, sem),
           pltpu.async_copy(x0.at[p_all.at[wid]], pb, sem),
           pltpu.async_copy(x0.at[n_all.at[wid]], nb, sem)]
    for cp in cps:
        cp.wait()

    def _rg(g, racc):
        rowi = iota + g * 16
        dimv = jnp.zeros((16,), jnp.int32)
        for d in range(64):
            uv = plsc.load_gather(ub, [rowi, dimv])
            pv = plsc.load_gather(pb, [rowi, dimv])
            nv = plsc.load_gather(nb, [rowi, dimv])
            racc = racc + uv * uv + pv * pv + nv * nv
            dimv = dimv + 1
        return racc

    racc = lax.fori_loop(0, 8, _rg, zero16)
    regv[0, pl.ds(0, 16)] = racc
    pltpu.sync_copy(regv, reg_out.at[wid])

    for xt in (x1, x2, x3):
        pltpu.sync_copy(xt.at[u_all.at[wid]], ub, add=True)
        pltpu.sync_copy(xt.at[p_all.at[wid]], pb, add=True)
        pltpu.sync_copy(xt.at[n_all.at[wid]], nb, add=True)

    def _dot(g, carry):
        rowi = iota + g * 16
        dimv = jnp.zeros((16,), jnp.int32)
        pacc = zero16
        nacc = zero16
        for d in range(64):
            uv = plsc.load_gather(ub, [rowi, dimv])
            pv = plsc.load_gather(pb, [rowi, dimv])
            nv = plsc.load_gather(nb, [rowi, dimv])
            pacc = pacc + uv * pv
            nacc = nacc + uv * nv
            dimv = dimv + 1
        diff_v[0, pl.ds(g * 16, 16)] = (pacc - nacc) * (1.0 / 16.0)
        return carry

    lax.fori_loop(0, 8, _dot, 0)
    pltpu.sync_copy(diff_v, diff_out.at[wid])


_score = pl.kernel(
    _score_body,
    out_type=(jax.ShapeDtypeStruct((32, 1, 128), jnp.float32),
              jax.ShapeDtypeStruct((32, 1, 16), jnp.float32)),
    mesh=plsc.VectorSubcoreMesh(core_axis_name="c", subcore_axis_name="s"),
    compiler_params=pltpu.CompilerParams(
        needs_layout_passes=False, use_tc_tiling_on_sc=False),
    scratch_types=[
        pltpu.VMEM((32, 128), jnp.int32),
        pltpu.VMEM((32, 128), jnp.int32),
        pltpu.VMEM((32, 128), jnp.int32),
        pltpu.VMEM((128, _D), jnp.float32),
        pltpu.VMEM((128, _D), jnp.float32),
        pltpu.VMEM((128, _D), jnp.float32),
        pltpu.VMEM((1, 128), jnp.float32),
        pltpu.VMEM((1, 16), jnp.float32),
        pltpu.SemaphoreType.DMA,
    ],
)


def _fin_body(diff_ref, reg_ref, loss_ref, bpr_ref):
    d = diff_ref[...]
    bpr = -jnp.mean(jax.nn.log_sigmoid(d))
    reg = jnp.sum(reg_ref[...]) * (1.0 / _BATCH)
    loss_ref[...] = jnp.full((1, 1), bpr + _DECAY * reg, jnp.float32)
    bpr_ref[...] = jnp.full((1, 1), bpr, jnp.float32)


def kernel(user_emb, item_emb, author_emb, edge_weight, users, pos_items,
           neg_items, edge_index):
    x0 = jnp.concatenate(
        [user_emb, item_emb, author_emb,
         jnp.zeros((_NPAD - _NN, _D), jnp.float32)], axis=0)

    pad = _EPAD - _E
    src = jnp.concatenate(
        [edge_index[0], (jnp.arange(pad, dtype=jnp.int32) * 37) % _NN])
    dst = jnp.concatenate(
        [edge_index[1], jnp.full((pad,), -1, jnp.int32)])
    w = jnp.concatenate([edge_weight, jnp.zeros((pad,), jnp.float32)])
    src2d = src.reshape(_EROWS, 128)
    dst2d = dst.reshape(_EROWS, 128)
    w2d = w.reshape(_EROWS, 128)

    x1 = _prop(x0, src2d, dst2d, w2d)
    x2 = _prop(x1, src2d, dst2d, w2d)
    x3 = _prop(x2, src2d, dst2d, w2d)

    u2d = users.astype(jnp.int32).reshape(32, 128)
    p2d = (pos_items.astype(jnp.int32) + _NU).reshape(32, 128)
    n2d = (neg_items.astype(jnp.int32) + _NU).reshape(32, 128)
    diff3d, regbuf = _score(x0, x1, x2, x3, u2d, p2d, n2d)

    loss_a, bpr_a = pl.pallas_call(
        _fin_body,
        out_shape=(jax.ShapeDtypeStruct((1, 1), jnp.float32),
                   jax.ShapeDtypeStruct((1, 1), jnp.float32)),
    )(diff3d, regbuf)
    return (loss_a[0, 0], bpr_a[0, 0])


# speedup vs baseline: 1.7572x; 1.7572x over previous
import functools

import jax
import jax.numpy as jnp
from jax import lax
from jax.experimental import pallas as pl
from jax.experimental.pallas import tpu as pltpu
from jax.experimental.pallas import tpu_sc as plsc

_NU = 25000
_NI = 20000
_NA = 5000
_NN = 50000
_E = 800000
_D = 64
_BATCH = 4096
_DECAY = 1e-4

_NC = 2
_NS = 16
_HALF = 25024
_NPAD = _NC * _HALF
_WROWS = _HALF // _NS
_CH = 128
_TCH = 392
_EROWS = _NS * _TCH
_EPAD = _EROWS * 128
_EROWS_ALLOC = _EROWS + 8
_EALLOC = _EROWS_ALLOC * 128


def _prop_body(x, srcr, dstr, wr, y, src_sp, dst_sp, w_sp, dloc_sp, wm_sp,
               rows0, rows1, zbuf, dzero, acc,
               gsem0, gsem1, ssem0, ssem1, isem):
    c = lax.axis_index("c")
    s = lax.axis_index("s")
    base = c * _HALF
    zero16 = jnp.zeros((16,), jnp.float32)
    iota = lax.iota(jnp.int32, 16)
    rowsb = (rows0, rows1)
    gsems = (gsem0, gsem1)
    ssems = (ssem0, ssem1)
    tbase = s * _TCH

    def _zb(r, carry):
        for q in range(4):
            zbuf[r, pl.ds(q * 16, 16)] = zero16
        return carry

    lax.fori_loop(0, _CH, _zb, 0)
    for q in range(8):
        dzero[0, pl.ds(q * 16, 16)] = jnp.zeros((16,), jnp.int32)
    for k in range(12):
        pltpu.sync_copy(zbuf, acc.at[pl.ds(s * _WROWS + k * _CH, _CH)])
    pltpu.sync_copy(zbuf.at[pl.ds(0, 28)],
                    acc.at[pl.ds(s * _WROWS + 12 * _CH, 28)])
    plsc.subcore_barrier()

    def _dlocwm(p):
        for jj in range(4):
            for g8 in range(8):
                off = g8 * 16
                d16 = dst_sp[p, jj, pl.ds(off, 16)]
                lidx = d16 - base
                m = (lidx >= 0) & (lidx < _HALF)
                wraw = w_sp[p, jj, pl.ds(off, 16)]
                wm_sp[p, jj, pl.ds(off, 16)] = jnp.where(m, wraw, 0.0)
                dloc_sp[p, jj, pl.ds(off, 16)] = jnp.where(m, lidx,
                                                           d16 & 16383)

    def _idx_load(p, sb, sync):
        row = tbase + sb * 4
        if sync:
            pltpu.sync_copy(srcr.at[pl.ds(row, 4)], src_sp.at[p])
            pltpu.sync_copy(dstr.at[pl.ds(row, 4)], dst_sp.at[p])
            pltpu.sync_copy(wr.at[pl.ds(row, 4)], w_sp.at[p])
        else:
            pltpu.async_copy(srcr.at[pl.ds(row, 4)], src_sp.at[p], isem)
            pltpu.async_copy(dstr.at[pl.ds(row, 4)], dst_sp.at[p], isem)
            pltpu.async_copy(wr.at[pl.ds(row, 4)], w_sp.at[p], isem)

    def _idx_wait(p):
        pltpu.make_async_copy(srcr.at[pl.ds(0, 4)], src_sp.at[p], isem).wait()
        pltpu.make_async_copy(dstr.at[pl.ds(0, 4)], dst_sp.at[p], isem).wait()
        pltpu.make_async_copy(wr.at[pl.ds(0, 4)], w_sp.at[p], isem).wait()

    _idx_load(0, 0, True)
    _dlocwm(0)
    _idx_load(1, 1, False)
    pltpu.async_copy(zbuf, acc.at[dzero.at[0]], ssem0, add=True)
    pltpu.async_copy(zbuf, acc.at[dzero.at[0]], ssem1, add=True)
    pltpu.async_copy(x.at[src_sp.at[0, 0]], rows0, gsem0)

    def _iter(t, carry):
        for su in range(2):
            p, pn = su, 1 - su
            sb = 2 * t + su
            for j in range(4):
                sg = j % 2
                so = 1 - sg
                rs = rowsb[sg]
                pltpu.make_async_copy(x.at[src_sp.at[p, j]], rs,
                                      gsems[sg]).wait()

                @plsc.parallel_loop(0, 8, unroll=1)
                def _scale(g, rs=rs, p=p, j=j):
                    w16 = wm_sp[p, j, pl.ds(g * 16, 16)]
                    rowi = iota + g * 16
                    dimv = jnp.zeros((16,), jnp.int32)
                    for d in range(64):
                        v = plsc.load_gather(rs, [rowi, dimv])
                        plsc.store_scatter(rs, [rowi, dimv], v * w16)
                        dimv = dimv + 1

                if j == 0:
                    pj, pjj = pn, 3
                else:
                    pj, pjj = p, j - 1
                pltpu.make_async_copy(rowsb[so],
                                      acc.at[dloc_sp.at[pj, pjj]],
                                      ssems[so]).wait()
                if j == 3:
                    _idx_wait(pn)
                    nref = src_sp.at[pn, 0]
                else:
                    nref = src_sp.at[p, j + 1]
                pltpu.async_copy(x.at[nref], rowsb[so], gsems[so])
                pltpu.async_copy(rs, acc.at[dloc_sp.at[p, j]],
                                 ssems[sg], add=True)
            _dlocwm(pn)
            _idx_load(p, sb + 2, False)
        return carry

    lax.fori_loop(0, 49, _iter, 0)

    pltpu.make_async_copy(x.at[src_sp.at[0, 0]], rows0, gsem0).wait()
    pltpu.make_async_copy(rows1, acc.at[dloc_sp.at[1, 3]], ssem1).wait()
    _idx_wait(1)
    plsc.subcore_barrier()
    pltpu.sync_copy(acc.at[pl.ds(s * _WROWS, _WROWS)],
                    y.at[pl.ds(c * _HALF + s * _WROWS, _WROWS)])


_prop = pl.kernel(
    _prop_body,
    out_type=jax.ShapeDtypeStruct((_NPAD, _D), jnp.float32),
    mesh=plsc.VectorSubcoreMesh(core_axis_name="c", subcore_axis_name="s"),
    compiler_params=pltpu.CompilerParams(
        needs_layout_passes=False, use_tc_tiling_on_sc=False),
    scratch_types=[
        pltpu.VMEM((2, 4, 128), jnp.int32),
        pltpu.VMEM((2, 4, 128), jnp.int32),
        pltpu.VMEM((2, 4, 128), jnp.float32),
        pltpu.VMEM((2, 4, 128), jnp.int32),
        pltpu.VMEM((2, 4, 128), jnp.float32),
        pltpu.VMEM((_CH, _D), jnp.float32),
        pltpu.VMEM((_CH, _D), jnp.float32),
        pltpu.VMEM((_CH, _D), jnp.float32),
        pltpu.VMEM((1, 128), jnp.int32),
        pltpu.VMEM_SHARED((_HALF, _D), jnp.float32),
        pltpu.SemaphoreType.DMA,
        pltpu.SemaphoreType.DMA,
        pltpu.SemaphoreType.DMA,
        pltpu.SemaphoreType.DMA,
        pltpu.SemaphoreType.DMA,
    ],
)


def _score_body(x0, x1, x2, x3, u2d, p2d, n2d, diff_out, reg_out,
                u_all, p_all, n_all, ub, pb, nb, diff_v, regv, sem):
    c = lax.axis_index("c")
    s = lax.axis_index("s")
    wid = s * _NC + c
    iota = lax.iota(jnp.int32, 16)
    zero16 = jnp.zeros((16,), jnp.float32)

    pltpu.sync_copy(u2d, u_all)
    pltpu.sync_copy(p2d, p_all)
    pltpu.sync_copy(n2d, n_all)
    cps = [pltpu.async_copy(x0.at[u_all.at[wid]], ub, sem),
           pltpu.async_copy(x0.at[p_all.at[wid]], pb, sem),
           pltpu.async_copy(x0.at[n_all.at[wid]], nb, sem)]
    for cp in cps:
        cp.wait()

    def _rg(g, racc):
        rowi = iota + g * 16
        dimv = jnp.zeros((16,), jnp.int32)
        for d in range(64):
            uv = plsc.load_gather(ub, [rowi, dimv])
            pv = plsc.load_gather(pb, [rowi, dimv])
            nv = plsc.load_gather(nb, [rowi, dimv])
            racc = racc + uv * uv + pv * pv + nv * nv
            dimv = dimv + 1
        return racc

    racc = lax.fori_loop(0, 8, _rg, zero16)
    regv[0, pl.ds(0, 16)] = racc
    pltpu.sync_copy(regv, reg_out.at[wid])

    for xt in (x1, x2, x3):
        pltpu.sync_copy(xt.at[u_all.at[wid]], ub, add=True)
        pltpu.sync_copy(xt.at[p_all.at[wid]], pb, add=True)
        pltpu.sync_copy(xt.at[n_all.at[wid]], nb, add=True)

    def _dot(g, carry):
        rowi = iota + g * 16
        dimv = jnp.zeros((16,), jnp.int32)
        pacc = zero16
        nacc = zero16
        for d in range(64):
            uv = plsc.load_gather(ub, [rowi, dimv])
            pv = plsc.load_gather(pb, [rowi, dimv])
            nv = plsc.load_gather(nb, [rowi, dimv])
            pacc = pacc + uv * pv
            nacc = nacc + uv * nv
            dimv = dimv + 1
        diff_v[0, pl.ds(g * 16, 16)] = (pacc - nacc) * (1.0 / 16.0)
        return carry

    lax.fori_loop(0, 8, _dot, 0)
    pltpu.sync_copy(diff_v, diff_out.at[wid])


_score = pl.kernel(
    _score_body,
    out_type=(jax.ShapeDtypeStruct((32, 1, 128), jnp.float32),
              jax.ShapeDtypeStruct((32, 1, 16), jnp.float32)),
    mesh=plsc.VectorSubcoreMesh(core_axis_name="c", subcore_axis_name="s"),
    compiler_params=pltpu.CompilerParams(
        needs_layout_passes=False, use_tc_tiling_on_sc=False),
    scratch_types=[
        pltpu.VMEM((32, 128), jnp.int32),
        pltpu.VMEM((32, 128), jnp.int32),
        pltpu.VMEM((32, 128), jnp.int32),
        pltpu.VMEM((128, _D), jnp.float32),
        pltpu.VMEM((128, _D), jnp.float32),
        pltpu.VMEM((128, _D), jnp.float32),
        pltpu.VMEM((1, 128), jnp.float32),
        pltpu.VMEM((1, 16), jnp.float32),
        pltpu.SemaphoreType.DMA,
    ],
)


def _fin_body(diff_ref, reg_ref, loss_ref, bpr_ref):
    d = diff_ref[...]
    bpr = -jnp.mean(jax.nn.log_sigmoid(d))
    reg = jnp.sum(reg_ref[...]) * (1.0 / _BATCH)
    loss_ref[...] = jnp.full((1, 1), bpr + _DECAY * reg, jnp.float32)
    bpr_ref[...] = jnp.full((1, 1), bpr, jnp.float32)


def kernel(user_emb, item_emb, author_emb, edge_weight, users, pos_items,
           neg_items, edge_index):
    x0 = jnp.concatenate(
        [user_emb, item_emb, author_emb,
         jnp.zeros((_NPAD - _NN, _D), jnp.float32)], axis=0)

    pad = _EALLOC - _E
    src = jnp.concatenate(
        [edge_index[0], (jnp.arange(pad, dtype=jnp.int32) * 37) % _NN])
    dst = jnp.concatenate(
        [edge_index[1], jnp.full((pad,), -1, jnp.int32)])
    w = jnp.concatenate([edge_weight, jnp.zeros((pad,), jnp.float32)])
    src2d = src.reshape(_EROWS_ALLOC, 128)
    dst2d = dst.reshape(_EROWS_ALLOC, 128)
    w2d = w.reshape(_EROWS_ALLOC, 128)

    x1 = _prop(x0, src2d, dst2d, w2d)
    x2 = _prop(x1, src2d, dst2d, w2d)
    x3 = _prop(x2, src2d, dst2d, w2d)

    u2d = users.astype(jnp.int32).reshape(32, 128)
    p2d = (pos_items.astype(jnp.int32) + _NU).reshape(32, 128)
    n2d = (neg_items.astype(jnp.int32) + _NU).reshape(32, 128)
    diff3d, regbuf = _score(x0, x1, x2, x3, u2d, p2d, n2d)

    loss_a, bpr_a = pl.pallas_call(
        _fin_body,
        out_shape=(jax.ShapeDtypeStruct((1, 1), jnp.float32),
                   jax.ShapeDtypeStruct((1, 1), jnp.float32)),
    )(diff3d, regbuf)
    return (loss_a[0, 0], bpr_a[0, 0])


# speedup vs baseline: 2.3103x; 1.3148x over previous
import functools

import jax
import jax.numpy as jnp
from jax import lax
from jax.experimental import pallas as pl
from jax.experimental.pallas import tpu as pltpu
from jax.experimental.pallas import tpu_sc as plsc

_NU = 25000
_NI = 20000
_NA = 5000
_NN = 50000
_E = 800000
_D = 64
_BATCH = 4096
_DECAY = 1e-4

_NC = 2
_NS = 16
_HALF = 25024
_NPAD = _NC * _HALF
_WROWS = _HALF // _NS
_CH = 128
_TCH = 392
_EROWS = _NS * _TCH
_EPAD = _EROWS * 128
_EROWS_ALLOC = _EROWS + 8
_EALLOC = _EROWS_ALLOC * 128


def _prop_body(x, srcr, dstr, wr, y, src_sp, dst_sp, w_sp, dloc_sp, wm_sp,
               rows0, rows1, zbuf, dzero, acc,
               gsem0, gsem1, ssem0, ssem1, isem):
    c = lax.axis_index("c")
    s = lax.axis_index("s")
    base = c * _HALF
    zero16 = jnp.zeros((16,), jnp.float32)
    iota = lax.iota(jnp.int32, 16)
    rowsb = (rows0, rows1)
    gsems = (gsem0, gsem1)
    ssems = (ssem0, ssem1)
    tbase = s * _TCH

    def _zb(r, carry):
        for q in range(4):
            zbuf[r, pl.ds(q * 16, 16)] = zero16
        return carry

    lax.fori_loop(0, _CH, _zb, 0)
    for q in range(8):
        dzero[0, pl.ds(q * 16, 16)] = jnp.zeros((16,), jnp.int32)
    for k in range(12):
        pltpu.sync_copy(zbuf, acc.at[pl.ds(s * _WROWS + k * _CH, _CH)])
    pltpu.sync_copy(zbuf.at[pl.ds(0, 28)],
                    acc.at[pl.ds(s * _WROWS + 12 * _CH, 28)])
    plsc.subcore_barrier()

    def _dlocwm(p):
        for jj in range(4):
            for g8 in range(8):
                off = g8 * 16
                d16 = dst_sp[p, jj, pl.ds(off, 16)]
                lidx = d16 - base
                m = (lidx >= 0) & (lidx < _HALF)
                wraw = w_sp[p, jj, pl.ds(off, 16)]
                wm_sp[p, jj, pl.ds(off, 16)] = jnp.where(m, wraw, 0.0)
                dloc_sp[p, jj, pl.ds(off, 16)] = jnp.where(m, lidx,
                                                           d16 & 16383)

    def _idx_load(p, sb, sync):
        row = tbase + sb * 4
        if sync:
            pltpu.sync_copy(srcr.at[pl.ds(row, 4)], src_sp.at[p])
            pltpu.sync_copy(dstr.at[pl.ds(row, 4)], dst_sp.at[p])
            pltpu.sync_copy(wr.at[pl.ds(row, 4)], w_sp.at[p])
        else:
            pltpu.async_copy(srcr.at[pl.ds(row, 4)], src_sp.at[p], isem)
            pltpu.async_copy(dstr.at[pl.ds(row, 4)], dst_sp.at[p], isem)
            pltpu.async_copy(wr.at[pl.ds(row, 4)], w_sp.at[p], isem)

    def _idx_wait(p):
        pltpu.make_async_copy(srcr.at[pl.ds(0, 4)], src_sp.at[p], isem).wait()
        pltpu.make_async_copy(dstr.at[pl.ds(0, 4)], dst_sp.at[p], isem).wait()
        pltpu.make_async_copy(wr.at[pl.ds(0, 4)], w_sp.at[p], isem).wait()

    _idx_load(0, 0, True)
    _dlocwm(0)
    _idx_load(1, 1, False)
    pltpu.async_copy(zbuf, acc.at[dzero.at[0]], ssem0, add=True)
    pltpu.async_copy(zbuf, acc.at[dzero.at[0]], ssem1, add=True)
    pltpu.async_copy(x.at[src_sp.at[0, 0]], rows0, gsem0)

    def _iter(t, carry):
        for su in range(2):
            p, pn = su, 1 - su
            sb = 2 * t + su
            for j in range(4):
                sg = j % 2
                so = 1 - sg
                rs = rowsb[sg]
                pltpu.make_async_copy(x.at[src_sp.at[p, j]], rs,
                                      gsems[sg]).wait()

                w16s = [wm_sp[p, j, pl.ds(g * 16, 16)] for g in range(8)]
                rowis = [iota + g * 16 for g in range(8)]

                @plsc.parallel_loop(0, 64, unroll=2)
                def _scale(d, rs=rs, w16s=w16s, rowis=rowis):
                    dimv = jnp.full((16,), d, jnp.int32)
                    for g in range(8):
                        v = plsc.load_gather(rs, [rowis[g], dimv])
                        plsc.store_scatter(rs, [rowis[g], dimv],
                                           v * w16s[g])

                if j == 0:
                    pj, pjj = pn, 3
                else:
                    pj, pjj = p, j - 1
                pltpu.make_async_copy(rowsb[so],
                                      acc.at[dloc_sp.at[pj, pjj]],
                                      ssems[so]).wait()
                if j == 3:
                    _idx_wait(pn)
                    nref = src_sp.at[pn, 0]
                else:
                    nref = src_sp.at[p, j + 1]
                pltpu.async_copy(x.at[nref], rowsb[so], gsems[so])
                pltpu.async_copy(rs, acc.at[dloc_sp.at[p, j]],
                                 ssems[sg], add=True)
            _dlocwm(pn)
            _idx_load(p, sb + 2, False)
        return carry

    lax.fori_loop(0, 49, _iter, 0)

    pltpu.make_async_copy(x.at[src_sp.at[0, 0]], rows0, gsem0).wait()
    pltpu.make_async_copy(rows1, acc.at[dloc_sp.at[1, 3]], ssem1).wait()
    _idx_wait(1)
    plsc.subcore_barrier()
    pltpu.sync_copy(acc.at[pl.ds(s * _WROWS, _WROWS)],
                    y.at[pl.ds(c * _HALF + s * _WROWS, _WROWS)])


_prop = pl.kernel(
    _prop_body,
    out_type=jax.ShapeDtypeStruct((_NPAD, _D), jnp.float32),
    mesh=plsc.VectorSubcoreMesh(core_axis_name="c", subcore_axis_name="s"),
    compiler_params=pltpu.CompilerParams(
        needs_layout_passes=False, use_tc_tiling_on_sc=False),
    scratch_types=[
        pltpu.VMEM((2, 4, 128), jnp.int32),
        pltpu.VMEM((2, 4, 128), jnp.int32),
        pltpu.VMEM((2, 4, 128), jnp.float32),
        pltpu.VMEM((2, 4, 128), jnp.int32),
        pltpu.VMEM((2, 4, 128), jnp.float32),
        pltpu.VMEM((_CH, _D), jnp.float32),
        pltpu.VMEM((_CH, _D), jnp.float32),
        pltpu.VMEM((_CH, _D), jnp.float32),
        pltpu.VMEM((1, 128), jnp.int32),
        pltpu.VMEM_SHARED((_HALF, _D), jnp.float32),
        pltpu.SemaphoreType.DMA,
        pltpu.SemaphoreType.DMA,
        pltpu.SemaphoreType.DMA,
        pltpu.SemaphoreType.DMA,
        pltpu.SemaphoreType.DMA,
    ],
)


def _score_body(x0, x1, x2, x3, u2d, p2d, n2d, diff_out, reg_out,
                u_all, p_all, n_all, ub, pb, nb, diff_v, regv, sem):
    c = lax.axis_index("c")
    s = lax.axis_index("s")
    wid = s * _NC + c
    iota = lax.iota(jnp.int32, 16)
    zero16 = jnp.zeros((16,), jnp.float32)

    pltpu.sync_copy(u2d, u_all)
    pltpu.sync_copy(p2d, p_all)
    pltpu.sync_copy(n2d, n_all)
    cps = [pltpu.async_copy(x0.at[u_all.at[wid]], ub, sem),
           pltpu.async_copy(x0.at[p_all.at[wid]], pb, sem),
           pltpu.async_copy(x0.at[n_all.at[wid]], nb, sem)]
    for cp in cps:
        cp.wait()

    def _rg(g, racc):
        rowi = iota + g * 16
        dimv = jnp.zeros((16,), jnp.int32)
        for d in range(64):
            uv = plsc.load_gather(ub, [rowi, dimv])
            pv = plsc.load_gather(pb, [rowi, dimv])
            nv = plsc.load_gather(nb, [rowi, dimv])
            racc = racc + uv * uv + pv * pv + nv * nv
            dimv = dimv + 1
        return racc

    racc = lax.fori_loop(0, 8, _rg, zero16)
    regv[0, pl.ds(0, 16)] = racc
    pltpu.sync_copy(regv, reg_out.at[wid])

    for xt in (x1, x2, x3):
        pltpu.sync_copy(xt.at[u_all.at[wid]], ub, add=True)
        pltpu.sync_copy(xt.at[p_all.at[wid]], pb, add=True)
        pltpu.sync_copy(xt.at[n_all.at[wid]], nb, add=True)

    def _dot(g, carry):
        rowi = iota + g * 16
        dimv = jnp.zeros((16,), jnp.int32)
        pacc = zero16
        nacc = zero16
        for d in range(64):
            uv = plsc.load_gather(ub, [rowi, dimv])
            pv = plsc.load_gather(pb, [rowi, dimv])
            nv = plsc.load_gather(nb, [rowi, dimv])
            pacc = pacc + uv * pv
            nacc = nacc + uv * nv
            dimv = dimv + 1
        diff_v[0, pl.ds(g * 16, 16)] = (pacc - nacc) * (1.0 / 16.0)
        return carry

    lax.fori_loop(0, 8, _dot, 0)
    pltpu.sync_copy(diff_v, diff_out.at[wid])


_score = pl.kernel(
    _score_body,
    out_type=(jax.ShapeDtypeStruct((32, 1, 128), jnp.float32),
              jax.ShapeDtypeStruct((32, 1, 16), jnp.float32)),
    mesh=plsc.VectorSubcoreMesh(core_axis_name="c", subcore_axis_name="s"),
    compiler_params=pltpu.CompilerParams(
        needs_layout_passes=False, use_tc_tiling_on_sc=False),
    scratch_types=[
        pltpu.VMEM((32, 128), jnp.int32),
        pltpu.VMEM((32, 128), jnp.int32),
        pltpu.VMEM((32, 128), jnp.int32),
        pltpu.VMEM((128, _D), jnp.float32),
        pltpu.VMEM((128, _D), jnp.float32),
        pltpu.VMEM((128, _D), jnp.float32),
        pltpu.VMEM((1, 128), jnp.float32),
        pltpu.VMEM((1, 16), jnp.float32),
        pltpu.SemaphoreType.DMA,
    ],
)


def _fin_body(diff_ref, reg_ref, loss_ref, bpr_ref):
    d = diff_ref[...]
    bpr = -jnp.mean(jax.nn.log_sigmoid(d))
    reg = jnp.sum(reg_ref[...]) * (1.0 / _BATCH)
    loss_ref[...] = jnp.full((1, 1), bpr + _DECAY * reg, jnp.float32)
    bpr_ref[...] = jnp.full((1, 1), bpr, jnp.float32)


def kernel(user_emb, item_emb, author_emb, edge_weight, users, pos_items,
           neg_items, edge_index):
    x0 = jnp.concatenate(
        [user_emb, item_emb, author_emb,
         jnp.zeros((_NPAD - _NN, _D), jnp.float32)], axis=0)

    pad = _EALLOC - _E
    src = jnp.concatenate(
        [edge_index[0], (jnp.arange(pad, dtype=jnp.int32) * 37) % _NN])
    dst = jnp.concatenate(
        [edge_index[1], jnp.full((pad,), -1, jnp.int32)])
    w = jnp.concatenate([edge_weight, jnp.zeros((pad,), jnp.float32)])
    src2d = src.reshape(_EROWS_ALLOC, 128)
    dst2d = dst.reshape(_EROWS_ALLOC, 128)
    w2d = w.reshape(_EROWS_ALLOC, 128)

    x1 = _prop(x0, src2d, dst2d, w2d)
    x2 = _prop(x1, src2d, dst2d, w2d)
    x3 = _prop(x2, src2d, dst2d, w2d)

    u2d = users.astype(jnp.int32).reshape(32, 128)
    p2d = (pos_items.astype(jnp.int32) + _NU).reshape(32, 128)
    n2d = (neg_items.astype(jnp.int32) + _NU).reshape(32, 128)
    diff3d, regbuf = _score(x0, x1, x2, x3, u2d, p2d, n2d)

    loss_a, bpr_a = pl.pallas_call(
        _fin_body,
        out_shape=(jax.ShapeDtypeStruct((1, 1), jnp.float32),
                   jax.ShapeDtypeStruct((1, 1), jnp.float32)),
    )(diff3d, regbuf)
    return (loss_a[0, 0], bpr_a[0, 0])


# speedup vs baseline: 5.6559x; 2.4481x over previous
import functools

import jax
import jax.numpy as jnp
from jax import lax
from jax.experimental import pallas as pl
from jax.experimental.pallas import tpu as pltpu
from jax.experimental.pallas import tpu_sc as plsc

_NU = 25000
_NI = 20000
_NA = 5000
_NN = 50000
_E = 800000
_D = 64
_BATCH = 4096
_DECAY = 1e-4

_NC = 2
_NS = 16
_HALF = 25024
_NPAD = _NC * _HALF
_WROWS = _HALF // _NS
_CH = 128
_TCH = 392
_EROWS = _NS * _TCH
_EPAD = _EROWS * 128
_EROWS_ALLOC = _EROWS + 8
_EALLOC = _EROWS_ALLOC * 128


def _prop_body(x, srcr, dstr, wr, y, src_sp, dst_sp, w_sp, dloc_sp, wm_sp,
               rows0, rows1, wexp, acc,
               gsem0, gsem1, ssem0, ssem1, isem):
    c = lax.axis_index("c")
    s = lax.axis_index("s")
    base = c * _HALF
    zero16 = jnp.zeros((16,), jnp.float32)
    iota = lax.iota(jnp.int32, 16)
    rowsb = (rows0, rows1)
    gsems = (gsem0, gsem1)
    ssems = (ssem0, ssem1)
    tbase = s * _TCH

    def _zb(r, carry):
        for q in range(4):
            rows0[r, pl.ds(q * 16, 16)] = zero16
        return carry

    lax.fori_loop(0, _CH, _zb, 0)
    for k in range(12):
        pltpu.sync_copy(rows0, acc.at[pl.ds(s * _WROWS + k * _CH, _CH)])
    pltpu.sync_copy(rows0.at[pl.ds(0, 28)],
                    acc.at[pl.ds(s * _WROWS + 12 * _CH, 28)])
    plsc.subcore_barrier()

    def _dlocwm(p):
        for jj in range(4):
            for g8 in range(8):
                off = g8 * 16
                d16 = dst_sp[p, jj, pl.ds(off, 16)]
                lidx = d16 - base
                m = (lidx >= 0) & (lidx < _HALF)
                wraw = w_sp[p, jj, pl.ds(off, 16)]
                wm_sp[p, jj, pl.ds(off, 16)] = jnp.where(m, wraw, 0.0)
                dloc_sp[p, jj, pl.ds(off, 16)] = jnp.where(m, lidx,
                                                           d16 & 16383)

    def _idx_load(p, sb, sync):
        row = tbase + sb * 4
        if sync:
            pltpu.sync_copy(srcr.at[pl.ds(row, 4)], src_sp.at[p])
            pltpu.sync_copy(dstr.at[pl.ds(row, 4)], dst_sp.at[p])
            pltpu.sync_copy(wr.at[pl.ds(row, 4)], w_sp.at[p])
        else:
            pltpu.async_copy(srcr.at[pl.ds(row, 4)], src_sp.at[p], isem)
            pltpu.async_copy(dstr.at[pl.ds(row, 4)], dst_sp.at[p], isem)
            pltpu.async_copy(wr.at[pl.ds(row, 4)], w_sp.at[p], isem)

    def _idx_wait(p):
        pltpu.make_async_copy(srcr.at[pl.ds(0, 4)], src_sp.at[p], isem).wait()
        pltpu.make_async_copy(dstr.at[pl.ds(0, 4)], dst_sp.at[p], isem).wait()
        pltpu.make_async_copy(wr.at[pl.ds(0, 4)], w_sp.at[p], isem).wait()

    _idx_load(0, 0, True)
    _dlocwm(0)
    _idx_load(1, 1, False)
    pltpu.async_copy(x.at[src_sp.at[0, 0]], rows0, gsem0)

    def _iter(t, carry):
        for su in range(2):
            p, pn = su, 1 - su
            sb = 2 * t + su
            for j in range(4):
                sg = j % 2
                so = 1 - sg
                rs = rowsb[sg]
                pltpu.make_async_copy(x.at[src_sp.at[p, j]], rs,
                                      gsems[sg]).wait()

                w16s = [wm_sp[p, j, pl.ds(g * 16, 16)] for g in range(8)]
                rowis = [iota + g * 16 for g in range(8)]

                @plsc.parallel_loop(0, 16, unroll=2)
                def _wx(l, w16s=w16s, rowis=rowis):
                    lv = jnp.full((16,), l, jnp.int32)
                    for g in range(8):
                        plsc.store_scatter(wexp, [rowis[g], lv], w16s[g])

                @plsc.parallel_loop(0, _CH, unroll=4)
                def _scale(e, rs=rs):
                    wv = wexp[e, pl.ds(0, 16)]
                    for q in range(4):
                        rs[e, pl.ds(q * 16, 16)] = (
                            rs[e, pl.ds(q * 16, 16)] * wv)

                if j == 0:
                    pj, pjj = pn, 3
                else:
                    pj, pjj = p, j - 1

                def _wait_prev_scatter(so=so, pj=pj, pjj=pjj):
                    pltpu.make_async_copy(rowsb[so],
                                          acc.at[dloc_sp.at[pj, pjj]],
                                          ssems[so]).wait()

                if su == 0 and j == 0:
                    pl.when(t > 0)(_wait_prev_scatter)
                else:
                    _wait_prev_scatter()
                if j == 3:
                    _idx_wait(pn)
                    nref = src_sp.at[pn, 0]
                else:
                    nref = src_sp.at[p, j + 1]
                pltpu.async_copy(x.at[nref], rowsb[so], gsems[so])
                pltpu.async_copy(rs, acc.at[dloc_sp.at[p, j]],
                                 ssems[sg], add=True)
            _dlocwm(pn)
            _idx_load(p, sb + 2, False)
        return carry

    lax.fori_loop(0, 49, _iter, 0)

    pltpu.make_async_copy(x.at[src_sp.at[0, 0]], rows0, gsem0).wait()
    pltpu.make_async_copy(rows1, acc.at[dloc_sp.at[1, 3]], ssem1).wait()
    _idx_wait(1)
    plsc.subcore_barrier()
    pltpu.sync_copy(acc.at[pl.ds(s * _WROWS, _WROWS)],
                    y.at[pl.ds(c * _HALF + s * _WROWS, _WROWS)])


_prop = pl.kernel(
    _prop_body,
    out_type=jax.ShapeDtypeStruct((_NPAD, _D), jnp.float32),
    mesh=plsc.VectorSubcoreMesh(core_axis_name="c", subcore_axis_name="s"),
    compiler_params=pltpu.CompilerParams(
        needs_layout_passes=False, use_tc_tiling_on_sc=False),
    scratch_types=[
        pltpu.VMEM((2, 4, 128), jnp.int32),
        pltpu.VMEM((2, 4, 128), jnp.int32),
        pltpu.VMEM((2, 4, 128), jnp.float32),
        pltpu.VMEM((2, 4, 128), jnp.int32),
        pltpu.VMEM((2, 4, 128), jnp.float32),
        pltpu.VMEM((_CH, _D), jnp.float32),
        pltpu.VMEM((_CH, _D), jnp.float32),
        pltpu.VMEM((_CH, 16), jnp.float32),
        pltpu.VMEM_SHARED((_HALF, _D), jnp.float32),
        pltpu.SemaphoreType.DMA,
        pltpu.SemaphoreType.DMA,
        pltpu.SemaphoreType.DMA,
        pltpu.SemaphoreType.DMA,
        pltpu.SemaphoreType.DMA,
    ],
)


def _score_body(x0, x1, x2, x3, u2d, p2d, n2d, diff_out, reg_out,
                u_all, p_all, n_all, ub, pb, nb, diff_v, regv, sem):
    c = lax.axis_index("c")
    s = lax.axis_index("s")
    wid = s * _NC + c
    iota = lax.iota(jnp.int32, 16)
    zero16 = jnp.zeros((16,), jnp.float32)

    pltpu.sync_copy(u2d, u_all)
    pltpu.sync_copy(p2d, p_all)
    pltpu.sync_copy(n2d, n_all)
    cps = [pltpu.async_copy(x0.at[u_all.at[wid]], ub, sem),
           pltpu.async_copy(x0.at[p_all.at[wid]], pb, sem),
           pltpu.async_copy(x0.at[n_all.at[wid]], nb, sem)]
    for cp in cps:
        cp.wait()

    def _rg(g, racc):
        rowi = iota + g * 16
        dimv = jnp.zeros((16,), jnp.int32)
        for d in range(64):
            uv = plsc.load_gather(ub, [rowi, dimv])
            pv = plsc.load_gather(pb, [rowi, dimv])
            nv = plsc.load_gather(nb, [rowi, dimv])
            racc = racc + uv * uv + pv * pv + nv * nv
            dimv = dimv + 1
        return racc

    racc = lax.fori_loop(0, 8, _rg, zero16)
    regv[0, pl.ds(0, 16)] = racc
    pltpu.sync_copy(regv, reg_out.at[wid])

    for xt in (x1, x2, x3):
        pltpu.sync_copy(xt.at[u_all.at[wid]], ub, add=True)
        pltpu.sync_copy(xt.at[p_all.at[wid]], pb, add=True)
        pltpu.sync_copy(xt.at[n_all.at[wid]], nb, add=True)

    def _dot(g, carry):
        rowi = iota + g * 16
        dimv = jnp.zeros((16,), jnp.int32)
        pacc = zero16
        nacc = zero16
        for d in range(64):
            uv = plsc.load_gather(ub, [rowi, dimv])
            pv = plsc.load_gather(pb, [rowi, dimv])
            nv = plsc.load_gather(nb, [rowi, dimv])
            pacc = pacc + uv * pv
            nacc = nacc + uv * nv
            dimv = dimv + 1
        diff_v[0, pl.ds(g * 16, 16)] = (pacc - nacc) * (1.0 / 16.0)
        return carry

    lax.fori_loop(0, 8, _dot, 0)
    pltpu.sync_copy(diff_v, diff_out.at[wid])


_score = pl.kernel(
    _score_body,
    out_type=(jax.ShapeDtypeStruct((32, 1, 128), jnp.float32),
              jax.ShapeDtypeStruct((32, 1, 16), jnp.float32)),
    mesh=plsc.VectorSubcoreMesh(core_axis_name="c", subcore_axis_name="s"),
    compiler_params=pltpu.CompilerParams(
        needs_layout_passes=False, use_tc_tiling_on_sc=False),
    scratch_types=[
        pltpu.VMEM((32, 128), jnp.int32),
        pltpu.VMEM((32, 128), jnp.int32),
        pltpu.VMEM((32, 128), jnp.int32),
        pltpu.VMEM((128, _D), jnp.float32),
        pltpu.VMEM((128, _D), jnp.float32),
        pltpu.VMEM((128, _D), jnp.float32),
        pltpu.VMEM((1, 128), jnp.float32),
        pltpu.VMEM((1, 16), jnp.float32),
        pltpu.SemaphoreType.DMA,
    ],
)


def _fin_body(diff_ref, reg_ref, loss_ref, bpr_ref):
    d = diff_ref[...]
    bpr = -jnp.mean(jax.nn.log_sigmoid(d))
    reg = jnp.sum(reg_ref[...]) * (1.0 / _BATCH)
    loss_ref[...] = jnp.full((1, 1), bpr + _DECAY * reg, jnp.float32)
    bpr_ref[...] = jnp.full((1, 1), bpr, jnp.float32)


def kernel(user_emb, item_emb, author_emb, edge_weight, users, pos_items,
           neg_items, edge_index):
    x0 = jnp.concatenate(
        [user_emb, item_emb, author_emb,
         jnp.zeros((_NPAD - _NN, _D), jnp.float32)], axis=0)

    pad = _EALLOC - _E
    src = jnp.concatenate(
        [edge_index[0], (jnp.arange(pad, dtype=jnp.int32) * 37) % _NN])
    dst = jnp.concatenate(
        [edge_index[1], jnp.full((pad,), -1, jnp.int32)])
    w = jnp.concatenate([edge_weight, jnp.zeros((pad,), jnp.float32)])
    src2d = src.reshape(_EROWS_ALLOC, 128)
    dst2d = dst.reshape(_EROWS_ALLOC, 128)
    w2d = w.reshape(_EROWS_ALLOC, 128)

    x1 = _prop(x0, src2d, dst2d, w2d)
    x2 = _prop(x1, src2d, dst2d, w2d)
    x3 = _prop(x2, src2d, dst2d, w2d)

    u2d = users.astype(jnp.int32).reshape(32, 128)
    p2d = (pos_items.astype(jnp.int32) + _NU).reshape(32, 128)
    n2d = (neg_items.astype(jnp.int32) + _NU).reshape(32, 128)
    diff3d, regbuf = _score(x0, x1, x2, x3, u2d, p2d, n2d)

    loss_a, bpr_a = pl.pallas_call(
        _fin_body,
        out_shape=(jax.ShapeDtypeStruct((1, 1), jnp.float32),
                   jax.ShapeDtypeStruct((1, 1), jnp.float32)),
    )(diff3d, regbuf)
    return (loss_a[0, 0], bpr_a[0, 0])


# speedup vs baseline: 8.6766x; 1.5341x over previous
import functools

import jax
import jax.numpy as jnp
from jax import lax
from jax.experimental import pallas as pl
from jax.experimental.pallas import tpu as pltpu
from jax.experimental.pallas import tpu_sc as plsc

_NU = 25000
_NI = 20000
_NA = 5000
_NN = 50000
_E = 800000
_D = 64
_BATCH = 4096
_DECAY = 1e-4

_NC = 2
_NS = 16
_HALF = 25024
_NPAD = _NC * _HALF
_WROWS = _HALF // _NS
_CH = 128
_TCH = 392
_EROWS = _NS * _TCH
_EPAD = _EROWS * 128
_EROWS_ALLOC = _EROWS + 8
_EALLOC = _EROWS_ALLOC * 128


def _prop_body(x, srcr, dstr, wr, y, src_sp, dst_sp, w_sp, dloc_sp, wm_sp,
               rows0, rows1, wexp, acc,
               gsem0, gsem1, ssem0, ssem1, isem):
    c = lax.axis_index("c")
    s = lax.axis_index("s")
    base = c * _HALF
    zero16 = jnp.zeros((16,), jnp.float32)
    iota = lax.iota(jnp.int32, 16)
    rowsb = (rows0, rows1)
    gsems = (gsem0, gsem1)
    ssems = (ssem0, ssem1)
    tbase = s * _TCH

    def _zb(r, carry):
        for q in range(4):
            rows0[r, pl.ds(q * 16, 16)] = zero16
        return carry

    lax.fori_loop(0, _CH, _zb, 0)
    for k in range(12):
        pltpu.sync_copy(rows0, acc.at[pl.ds(s * _WROWS + k * _CH, _CH)])
    pltpu.sync_copy(rows0.at[pl.ds(0, 28)],
                    acc.at[pl.ds(s * _WROWS + 12 * _CH, 28)])
    plsc.subcore_barrier()

    def _dlocwm(p):
        for jj in range(4):
            for g8 in range(8):
                off = g8 * 16
                d16 = dst_sp[p, jj, pl.ds(off, 16)]
                lidx = d16 - base
                m = (lidx >= 0) & (lidx < _HALF)
                wraw = w_sp[p, jj, pl.ds(off, 16)]
                wm_sp[p, jj, pl.ds(off, 16)] = jnp.where(m, wraw, 0.0)
                dloc_sp[p, jj, pl.ds(off, 16)] = jnp.where(m, lidx,
                                                           d16 & 16383)

    def _idx_load(p, sb, sync):
        row = tbase + sb * 4
        if sync:
            pltpu.sync_copy(srcr.at[pl.ds(row, 4)], src_sp.at[p])
            pltpu.sync_copy(dstr.at[pl.ds(row, 4)], dst_sp.at[p])
            pltpu.sync_copy(wr.at[pl.ds(row, 4)], w_sp.at[p])
        else:
            pltpu.async_copy(srcr.at[pl.ds(row, 4)], src_sp.at[p], isem)
            pltpu.async_copy(dstr.at[pl.ds(row, 4)], dst_sp.at[p], isem)
            pltpu.async_copy(wr.at[pl.ds(row, 4)], w_sp.at[p], isem)

    def _idx_wait(p):
        pltpu.make_async_copy(srcr.at[pl.ds(0, 4)], src_sp.at[p], isem).wait()
        pltpu.make_async_copy(dstr.at[pl.ds(0, 4)], dst_sp.at[p], isem).wait()
        pltpu.make_async_copy(wr.at[pl.ds(0, 4)], w_sp.at[p], isem).wait()

    _idx_load(0, 0, True)
    _dlocwm(0)
    _idx_load(1, 1, False)
    pltpu.async_copy(x.at[src_sp.at[0, 0]], rows0, gsem0)

    def _iter(t, carry):
        for su in range(2):
            p, pn = su, 1 - su
            sb = 2 * t + su
            for j in range(4):
                sg = j % 2
                so = 1 - sg
                rs = rowsb[sg]
                w16s = [wm_sp[p, j, pl.ds(g * 16, 16)] for g in range(8)]
                rowis = [iota + g * 16 for g in range(8)]

                @plsc.parallel_loop(0, 16, unroll=2)
                def _wx(l, w16s=w16s, rowis=rowis):
                    lv = jnp.full((16,), l, jnp.int32)
                    for g in range(8):
                        plsc.store_scatter(wexp, [rowis[g], lv], w16s[g])

                if j == 0:
                    pj, pjj = pn, 3
                else:
                    pj, pjj = p, j - 1

                def _wait_prev_scatter(so=so, pj=pj, pjj=pjj):
                    pltpu.make_async_copy(rowsb[so],
                                          acc.at[dloc_sp.at[pj, pjj]],
                                          ssems[so]).wait()

                if su == 0 and j == 0:
                    pl.when(t > 0)(_wait_prev_scatter)
                else:
                    _wait_prev_scatter()
                if j == 3:
                    _idx_wait(pn)
                    nref = src_sp.at[pn, 0]
                else:
                    nref = src_sp.at[p, j + 1]
                pltpu.async_copy(x.at[nref], rowsb[so], gsems[so])

                pltpu.make_async_copy(x.at[src_sp.at[p, j]], rs,
                                      gsems[sg]).wait()

                @plsc.parallel_loop(0, _CH, unroll=4)
                def _scale(e, rs=rs):
                    wv = wexp[e, pl.ds(0, 16)]
                    for q in range(4):
                        rs[e, pl.ds(q * 16, 16)] = (
                            rs[e, pl.ds(q * 16, 16)] * wv)

                pltpu.async_copy(rs, acc.at[dloc_sp.at[p, j]],
                                 ssems[sg], add=True)
            _dlocwm(pn)
            _idx_load(p, sb + 2, False)
        return carry

    lax.fori_loop(0, 49, _iter, 0)

    pltpu.make_async_copy(x.at[src_sp.at[0, 0]], rows0, gsem0).wait()
    pltpu.make_async_copy(rows1, acc.at[dloc_sp.at[1, 3]], ssem1).wait()
    _idx_wait(1)
    plsc.subcore_barrier()
    pltpu.sync_copy(acc.at[pl.ds(s * _WROWS, _WROWS)],
                    y.at[pl.ds(c * _HALF + s * _WROWS, _WROWS)])


_prop = pl.kernel(
    _prop_body,
    out_type=jax.ShapeDtypeStruct((_NPAD, _D), jnp.float32),
    mesh=plsc.VectorSubcoreMesh(core_axis_name="c", subcore_axis_name="s"),
    compiler_params=pltpu.CompilerParams(
        needs_layout_passes=False, use_tc_tiling_on_sc=False),
    scratch_types=[
        pltpu.VMEM((2, 4, 128), jnp.int32),
        pltpu.VMEM((2, 4, 128), jnp.int32),
        pltpu.VMEM((2, 4, 128), jnp.float32),
        pltpu.VMEM((2, 4, 128), jnp.int32),
        pltpu.VMEM((2, 4, 128), jnp.float32),
        pltpu.VMEM((_CH, _D), jnp.float32),
        pltpu.VMEM((_CH, _D), jnp.float32),
        pltpu.VMEM((_CH, 16), jnp.float32),
        pltpu.VMEM_SHARED((_HALF, _D), jnp.float32),
        pltpu.SemaphoreType.DMA,
        pltpu.SemaphoreType.DMA,
        pltpu.SemaphoreType.DMA,
        pltpu.SemaphoreType.DMA,
        pltpu.SemaphoreType.DMA,
    ],
)


def _score_body(x0, x1, x2, x3, u2d, p2d, n2d, diff_out, reg_out,
                u_all, p_all, n_all, ub, pb, nb, diff_v, regv, sem):
    c = lax.axis_index("c")
    s = lax.axis_index("s")
    wid = s * _NC + c
    iota = lax.iota(jnp.int32, 16)
    zero16 = jnp.zeros((16,), jnp.float32)

    pltpu.sync_copy(u2d, u_all)
    pltpu.sync_copy(p2d, p_all)
    pltpu.sync_copy(n2d, n_all)
    cps = [pltpu.async_copy(x0.at[u_all.at[wid]], ub, sem),
           pltpu.async_copy(x0.at[p_all.at[wid]], pb, sem),
           pltpu.async_copy(x0.at[n_all.at[wid]], nb, sem)]
    for cp in cps:
        cp.wait()

    def _rg(g, racc):
        rowi = iota + g * 16
        dimv = jnp.zeros((16,), jnp.int32)
        for d in range(64):
            uv = plsc.load_gather(ub, [rowi, dimv])
            pv = plsc.load_gather(pb, [rowi, dimv])
            nv = plsc.load_gather(nb, [rowi, dimv])
            racc = racc + uv * uv + pv * pv + nv * nv
            dimv = dimv + 1
        return racc

    racc = lax.fori_loop(0, 8, _rg, zero16)
    regv[0, pl.ds(0, 16)] = racc
    pltpu.sync_copy(regv, reg_out.at[wid])

    for xt in (x1, x2, x3):
        pltpu.sync_copy(xt.at[u_all.at[wid]], ub, add=True)
        pltpu.sync_copy(xt.at[p_all.at[wid]], pb, add=True)
        pltpu.sync_copy(xt.at[n_all.at[wid]], nb, add=True)

    def _dot(g, carry):
        rowi = iota + g * 16
        dimv = jnp.zeros((16,), jnp.int32)
        pacc = zero16
        nacc = zero16
        for d in range(64):
            uv = plsc.load_gather(ub, [rowi, dimv])
            pv = plsc.load_gather(pb, [rowi, dimv])
            nv = plsc.load_gather(nb, [rowi, dimv])
            pacc = pacc + uv * pv
            nacc = nacc + uv * nv
            dimv = dimv + 1
        diff_v[0, pl.ds(g * 16, 16)] = (pacc - nacc) * (1.0 / 16.0)
        return carry

    lax.fori_loop(0, 8, _dot, 0)
    pltpu.sync_copy(diff_v, diff_out.at[wid])


_score = pl.kernel(
    _score_body,
    out_type=(jax.ShapeDtypeStruct((32, 1, 128), jnp.float32),
              jax.ShapeDtypeStruct((32, 1, 16), jnp.float32)),
    mesh=plsc.VectorSubcoreMesh(core_axis_name="c", subcore_axis_name="s"),
    compiler_params=pltpu.CompilerParams(
        needs_layout_passes=False, use_tc_tiling_on_sc=False),
    scratch_types=[
        pltpu.VMEM((32, 128), jnp.int32),
        pltpu.VMEM((32, 128), jnp.int32),
        pltpu.VMEM((32, 128), jnp.int32),
        pltpu.VMEM((128, _D), jnp.float32),
        pltpu.VMEM((128, _D), jnp.float32),
        pltpu.VMEM((128, _D), jnp.float32),
        pltpu.VMEM((1, 128), jnp.float32),
        pltpu.VMEM((1, 16), jnp.float32),
        pltpu.SemaphoreType.DMA,
    ],
)


def _fin_body(diff_ref, reg_ref, loss_ref, bpr_ref):
    d = diff_ref[...]
    bpr = -jnp.mean(jax.nn.log_sigmoid(d))
    reg = jnp.sum(reg_ref[...]) * (1.0 / _BATCH)
    loss_ref[...] = jnp.full((1, 1), bpr + _DECAY * reg, jnp.float32)
    bpr_ref[...] = jnp.full((1, 1), bpr, jnp.float32)


def kernel(user_emb, item_emb, author_emb, edge_weight, users, pos_items,
           neg_items, edge_index):
    x0 = jnp.concatenate(
        [user_emb, item_emb, author_emb,
         jnp.zeros((_NPAD - _NN, _D), jnp.float32)], axis=0)

    pad = _EALLOC - _E
    src = jnp.concatenate(
        [edge_index[0], (jnp.arange(pad, dtype=jnp.int32) * 37) % _NN])
    dst = jnp.concatenate(
        [edge_index[1], jnp.full((pad,), -1, jnp.int32)])
    w = jnp.concatenate([edge_weight, jnp.zeros((pad,), jnp.float32)])
    src2d = src.reshape(_EROWS_ALLOC, 128)
    dst2d = dst.reshape(_EROWS_ALLOC, 128)
    w2d = w.reshape(_EROWS_ALLOC, 128)

    x1 = _prop(x0, src2d, dst2d, w2d)
    x2 = _prop(x1, src2d, dst2d, w2d)
    x3 = _prop(x2, src2d, dst2d, w2d)

    u2d = users.astype(jnp.int32).reshape(32, 128)
    p2d = (pos_items.astype(jnp.int32) + _NU).reshape(32, 128)
    n2d = (neg_items.astype(jnp.int32) + _NU).reshape(32, 128)
    diff3d, regbuf = _score(x0, x1, x2, x3, u2d, p2d, n2d)

    loss_a, bpr_a = pl.pallas_call(
        _fin_body,
        out_shape=(jax.ShapeDtypeStruct((1, 1), jnp.float32),
                   jax.ShapeDtypeStruct((1, 1), jnp.float32)),
    )(diff3d, regbuf)
    return (loss_a[0, 0], bpr_a[0, 0])
